# baseline (speedup 1.0000x reference)
import numpy as np

import jax
import jax.numpy as jnp
from jax import lax
from jax.experimental import pallas as pl
from jax.experimental.pallas import tpu as pltpu
from jax.experimental.pallas import tpu_sc as plsc

VOCAB = 100000
SEQ_LEN = 2048
DIM = 768
BATCH = 4
N = 10000

NUM_CORES = 2
NUM_SUBCORES = 16
NW = NUM_CORES * NUM_SUBCORES
SCHUNK = 16
NCHUNK = SEQ_LEN // SCHUNK
CHUNKS_PER_W = NCHUNK // NW
LANES = 16
VECS_PER_ROW = DIM // LANES


def _pos_factors():
    j = np.arange(DIM, dtype=np.float64)
    f = np.power(float(N), -2.0 * j / DIM)
    even = (np.arange(DIM) % 2) == 0
    c = np.arange(NCHUNK, dtype=np.float64)[:, None]
    a = (SCHUNK * c) * f[None, :]
    p = np.where(even, np.sin(a), np.cos(a))
    q = np.where(even, np.cos(a), -np.sin(a))
    pq = np.stack([p, q], axis=1).reshape(2 * NCHUNK, DIM)
    r = np.arange(SCHUNK, dtype=np.float64)[:, None]
    b = r * f[None, :]
    rr = np.stack([np.cos(b), np.sin(b)], axis=1).reshape(2 * SCHUNK, DIM)
    return (jnp.asarray(pq.astype(np.float32)),
            jnp.asarray(rr.astype(np.float32)))


def _sc_body(table_hbm, x_hbm, pq_hbm, rr_hbm, out_hbm,
             idx_all, pq_v, rr_v,
             r00, r10, r20, r30, r01, r11, r21, r31,
             isem, psem,
             g00, g10, g20, g30, g01, g11, g21, g31,
             o00, o10, o20, o30, o01, o11, o21, o31):
    wid = lax.axis_index("s") * NUM_CORES + lax.axis_index("c")
    rows = ((r00, r10, r20, r30), (r01, r11, r21, r31))
    gsem = ((g00, g10, g20, g30), (g01, g11, g21, g31))
    osem = ((o00, o10, o20, o30), (o01, o11, o21, o31))

    def chunk_s0(k):
        return (wid * CHUNKS_PER_W + k) * SCHUNK

    cps = [pltpu.async_copy(
        pq_hbm.at[pl.ds(2 * wid * CHUNKS_PER_W, 2 * CHUNKS_PER_W)], pq_v, psem),
        pltpu.async_copy(rr_hbm.at[:], rr_v, psem)]
    for k in range(CHUNKS_PER_W):
        s0 = chunk_s0(k)
        for b in range(BATCH):
            cps.append(pltpu.async_copy(
                x_hbm.at[b, pl.ds(s0, SCHUNK)], idx_all.at[k * BATCH + b], isem))
    for cp in cps:
        cp.wait()

    def fire_gathers(k):
        p = k % 2
        return [pltpu.async_copy(
            table_hbm.at[idx_all.at[k * BATCH + b]], rows[p][b], gsem[p][b])
            for b in range(BATCH)]

    gathers = [None, None]
    out_writes = [None, None]
    gathers[0] = fire_gathers(0)
    gathers[1] = fire_gathers(1)

    for k in range(CHUNKS_PER_W):
        p = k % 2
        for cp in gathers[p]:
            cp.wait()
        rv = rows[p]

        def j_body(j, _, k=k, rv=rv):
            slj = pl.ds(j * LANES, LANES)
            pj = pq_v[2 * k, slj]
            qj = pq_v[2 * k + 1, slj]
            for r in range(SCHUNK):
                t = pj * rr_v[2 * r, slj] + qj * rr_v[2 * r + 1, slj]
                for b in range(BATCH):
                    plsc.addupdate(rv[b].at[r, slj], t)
            return 0

        lax.fori_loop(0, VECS_PER_ROW, j_body, 0)

        s0 = chunk_s0(k)
        out_writes[p] = [pltpu.async_copy(
            rv[b], out_hbm.at[pl.ds(b * SEQ_LEN + s0, SCHUNK)], osem[p][b])
            for b in range(BATCH)]
        if k + 2 < CHUNKS_PER_W:
            for cp in out_writes[p]:
                cp.wait()
            out_writes[p] = None
            gathers[p] = fire_gathers(k + 2)
    for ow in out_writes:
        if ow is not None:
            for cp in ow:
                cp.wait()


def kernel(x, token_table):
    pq, rr = _pos_factors()
    x32 = x.astype(jnp.int32)
    mesh = plsc.VectorSubcoreMesh(core_axis_name="c", subcore_axis_name="s")
    out = pl.kernel(
        _sc_body,
        mesh=mesh,
        out_type=jax.ShapeDtypeStruct((BATCH * SEQ_LEN, DIM), jnp.float32),
        scratch_types=[
            pltpu.VMEM((CHUNKS_PER_W * BATCH, SCHUNK), jnp.int32),
            pltpu.VMEM((2 * CHUNKS_PER_W, DIM), jnp.float32),
            pltpu.VMEM((2 * SCHUNK, DIM), jnp.float32),
        ] + [pltpu.VMEM((SCHUNK, DIM), jnp.float32)] * 8
          + [pltpu.SemaphoreType.DMA] * 18,
    )(token_table, x32, pq, rr)
    return out.reshape(BATCH, SEQ_LEN, DIM)


# speedup vs baseline: 1.1025x; 1.1025x over previous
import numpy as np

import jax
import jax.numpy as jnp
from jax import lax
from jax.experimental import pallas as pl
from jax.experimental.pallas import tpu as pltpu
from jax.experimental.pallas import tpu_sc as plsc

VOCAB = 100000
SEQ_LEN = 2048
DIM = 768
BATCH = 4
N = 10000

NUM_CORES = 2
NUM_SUBCORES = 16
NW = NUM_CORES * NUM_SUBCORES
SCHUNK = 16
NCHUNK = SEQ_LEN // SCHUNK
CHUNKS_PER_W = NCHUNK // NW
LANES = 16
VECS_PER_ROW = DIM // LANES


def _positional_table():
    positions = np.arange(0, SEQ_LEN, dtype=np.float32)[:, None]
    den_even = np.power(float(N), 2.0 * np.arange(0, DIM, 2, dtype=np.float32) / DIM)
    den_odd = np.power(float(N), 2.0 * np.arange(1, DIM, 2, dtype=np.float32) / DIM)
    emb = np.zeros((SEQ_LEN, DIM), dtype=np.float32)
    emb[:, 0::2] = np.sin(positions / den_even)
    emb[:, 1::2] = np.cos(positions / den_odd)
    return jnp.asarray(emb)


def _sc_body(table_hbm, x_hbm, pos_hbm, out_hbm,
             idx_all, pos0, pos1,
             r00, r10, r20, r30, r01, r11, r21, r31,
             isem, ppsem0, ppsem1,
             g00, g10, g20, g30, g01, g11, g21, g31,
             o00, o10, o20, o30, o01, o11, o21, o31):
    wid = lax.axis_index("s") * NUM_CORES + lax.axis_index("c")
    pos_v = (pos0, pos1)
    ppsem = (ppsem0, ppsem1)
    rows = ((r00, r10, r20, r30), (r01, r11, r21, r31))
    gsem = ((g00, g10, g20, g30), (g01, g11, g21, g31))
    osem = ((o00, o10, o20, o30), (o01, o11, o21, o31))

    def chunk_s0(k):
        return (wid * CHUNKS_PER_W + k) * SCHUNK

    cps = []
    for k in range(CHUNKS_PER_W):
        s0 = chunk_s0(k)
        for b in range(BATCH):
            cps.append(pltpu.async_copy(
                x_hbm.at[b, pl.ds(s0, SCHUNK)], idx_all.at[k * BATCH + b], isem))
    pos_cp = [None, None]
    for k in range(2):
        pos_cp[k] = pltpu.async_copy(
            pos_hbm.at[pl.ds(chunk_s0(k), SCHUNK)], pos_v[k], ppsem[k])
    for cp in cps:
        cp.wait()

    def fire_gathers(k):
        p = k % 2
        return [pltpu.async_copy(
            table_hbm.at[idx_all.at[k * BATCH + b]], rows[p][b], gsem[p][b])
            for b in range(BATCH)]

    gathers = [None, None]
    out_writes = [None, None]
    gathers[0] = fire_gathers(0)
    gathers[1] = fire_gathers(1)

    for k in range(CHUNKS_PER_W):
        p = k % 2
        for cp in gathers[p]:
            cp.wait()
        pos_cp[p].wait()
        rv = rows[p]
        pv = pos_v[p]

        def row_add(r, _, rv=rv, pv=pv):
            for j in range(VECS_PER_ROW):
                sl = pl.ds(j * LANES, LANES)
                v = pv[r, sl]
                for b in range(BATCH):
                    plsc.addupdate(rv[b].at[r, sl], v)
            return 0

        lax.fori_loop(0, SCHUNK, row_add, 0)

        if k + 2 < CHUNKS_PER_W:
            pos_cp[p] = pltpu.async_copy(
                pos_hbm.at[pl.ds(chunk_s0(k + 2), SCHUNK)], pos_v[p], ppsem[p])
        s0 = chunk_s0(k)
        out_writes[p] = [pltpu.async_copy(
            rv[b], out_hbm.at[pl.ds(b * SEQ_LEN + s0, SCHUNK)], osem[p][b])
            for b in range(BATCH)]
        if k + 2 < CHUNKS_PER_W:
            for cp in out_writes[p]:
                cp.wait()
            out_writes[p] = None
            gathers[p] = fire_gathers(k + 2)
    for ow in out_writes:
        if ow is not None:
            for cp in ow:
                cp.wait()


def kernel(x, token_table):
    pos = _positional_table()
    x32 = x.astype(jnp.int32)
    mesh = plsc.VectorSubcoreMesh(core_axis_name="c", subcore_axis_name="s")
    out = pl.kernel(
        _sc_body,
        mesh=mesh,
        out_type=jax.ShapeDtypeStruct((BATCH * SEQ_LEN, DIM), jnp.float32),
        scratch_types=[
            pltpu.VMEM((CHUNKS_PER_W * BATCH, SCHUNK), jnp.int32),
            pltpu.VMEM((SCHUNK, DIM), jnp.float32),
            pltpu.VMEM((SCHUNK, DIM), jnp.float32),
        ] + [pltpu.VMEM((SCHUNK, DIM), jnp.float32)] * 8
          + [pltpu.SemaphoreType.DMA] * 19,
    )(token_table, x32, pos)
    return out.reshape(BATCH, SEQ_LEN, DIM)
